# baseline (speedup 1.0000x reference)
import numpy as np
import jax
import jax.numpy as jnp
from jax.experimental import pallas as pl
from jax.experimental.pallas import tpu as pltpu

N_DET = 5000
M_REF = 1000
C_IN = 256
SPATIAL = 49
FLAT = C_IN * SPATIAL
FC_OUT = 1024

BN = 200
BM = 200


def _pool_matrix():
    p = np.kron(np.eye(C_IN, dtype=np.float32), np.ones((SPATIAL, 1), np.float32))
    return jnp.asarray(p, dtype=jnp.bfloat16)


def _pool_fc(xb, p, w1, b1, w2, b2):
    hi = xb.astype(jnp.bfloat16)
    lo = (xb - hi.astype(jnp.float32)).astype(jnp.bfloat16)
    psum = (jnp.dot(hi, p, preferred_element_type=jnp.float32)
            + jnp.dot(lo, p, preferred_element_type=jnp.float32))
    pooled = psum / 49.0
    h = jnp.maximum(jnp.dot(pooled, w1, preferred_element_type=jnp.float32) + b1, 0.0)
    return jnp.dot(h, w2, preferred_element_type=jnp.float32) + b2


def _ref_kernel(rx_ref, p_ref, w1_ref, b1_ref, w2_ref, b2_ref, rf_ref):
    rf_ref[...] = _pool_fc(rx_ref[...], p_ref[...], w1_ref[...], b1_ref[...],
                           w2_ref[...], b2_ref[...])


def _main_kernel(x_ref, bb_ref, p_ref, w1_ref, b1_ref, w2_ref, b2_ref,
                 rft_ref, refg_ref, rcil_ref,
                 prod_ref, ious_ref, dxy_ref, dsp_ref):
    xf = _pool_fc(x_ref[...], p_ref[...], w1_ref[...], b1_ref[...],
                  w2_ref[...], b2_ref[...])
    prod_ref[...] = jnp.dot(xf, rft_ref[...], preferred_element_type=jnp.float32)

    bb = bb_ref[...]
    x1 = bb[:, 0:1]
    y1 = bb[:, 1:2]
    x2 = bb[:, 2:3]
    y2 = bb[:, 3:4]
    cx = (x1 + x2) / 2.0
    cy = (y1 + y2) / 2.0
    rg = refg_ref[...]
    rx1 = rg[0:1, :]
    ry1 = rg[1:2, :]
    rx2 = rg[2:3, :]
    ry2 = rg[3:4, :]
    rcx = rg[4:5, :]
    rcy = rg[5:6, :]
    areab = rg[6:7, :]
    dx = rcx - cx
    dy = rcy - cy
    sx1 = x1 + dx
    sy1 = y1 + dy
    sx2 = x2 + dx
    sy2 = y2 + dy
    wx = jnp.maximum(jnp.minimum(sx2, rx2) - jnp.maximum(sx1, rx1), 0.0)
    wy = jnp.maximum(jnp.minimum(sy2, ry2) - jnp.maximum(sy1, ry1), 0.0)
    ov = wx * wy
    areaa = (sx2 - sx1) * (sy2 - sy1)
    union = areaa + areab - ov
    ious_ref[...] = ov / jnp.maximum(union, 1e-6)

    col = jax.lax.broadcasted_iota(jnp.int32, (BN, M_REF + 1), 1)
    keep = col >= 1
    dxy_ref[0] = jnp.where(keep, dx, 0.0)
    dxy_ref[1] = jnp.where(keep, dy, 0.0)

    col2 = jax.lax.broadcasted_iota(jnp.int32, (BN, 2 * M_REF + 2), 1)
    c_il = jnp.where((col2 & 1) == 0, cx, cy)
    dsp_ref[0] = jnp.where(col2 >= 2, rcil_ref[...] - c_il, 0.0)


def kernel(bboxes, ref_bboxes, x, ref_x, x_n, ref_x_n, W1, b1, W2, b2):
    del x_n, ref_x_n
    x2 = x.reshape(N_DET, FLAT)
    rx2 = ref_x.reshape(M_REF, FLAT)
    p = _pool_matrix()
    b1r = b1.reshape(1, FC_OUT)
    b2r = b2.reshape(1, FC_OUT)

    rf = pl.pallas_call(
        _ref_kernel,
        grid=(M_REF // BM,),
        in_specs=[
            pl.BlockSpec((BM, FLAT), lambda i: (i, 0)),
            pl.BlockSpec((FLAT, C_IN), lambda i: (0, 0)),
            pl.BlockSpec((C_IN, FC_OUT), lambda i: (0, 0)),
            pl.BlockSpec((1, FC_OUT), lambda i: (0, 0)),
            pl.BlockSpec((FC_OUT, FC_OUT), lambda i: (0, 0)),
            pl.BlockSpec((1, FC_OUT), lambda i: (0, 0)),
        ],
        out_specs=pl.BlockSpec((BM, FC_OUT), lambda i: (i, 0)),
        out_shape=jax.ShapeDtypeStruct((M_REF, FC_OUT), jnp.float32),
        compiler_params=pltpu.CompilerParams(dimension_semantics=("arbitrary",)),
    )(rx2, p, W1, b1r, W2, b2r)
    rft = rf.T

    rcx = (ref_bboxes[:, 0] + ref_bboxes[:, 2]) / 2.0
    rcy = (ref_bboxes[:, 1] + ref_bboxes[:, 3]) / 2.0
    areab = (ref_bboxes[:, 2] - ref_bboxes[:, 0]) * (ref_bboxes[:, 3] - ref_bboxes[:, 1])
    refg = jnp.zeros((8, M_REF + 1), jnp.float32)
    refg = refg.at[0, 1:].set(ref_bboxes[:, 0])
    refg = refg.at[1, 1:].set(ref_bboxes[:, 1])
    refg = refg.at[2, 1:].set(ref_bboxes[:, 2])
    refg = refg.at[3, 1:].set(ref_bboxes[:, 3])
    refg = refg.at[4, 1:].set(rcx)
    refg = refg.at[5, 1:].set(rcy)
    refg = refg.at[6, 1:].set(areab)
    rcil = jnp.concatenate(
        [jnp.zeros((2,), jnp.float32), jnp.stack([rcx, rcy], axis=1).reshape(-1)]
    ).reshape(1, 2 * M_REF + 2)

    prod, ious2, dxy, dsp = pl.pallas_call(
        _main_kernel,
        grid=(N_DET // BN,),
        in_specs=[
            pl.BlockSpec((BN, FLAT), lambda i: (i, 0)),
            pl.BlockSpec((BN, 4), lambda i: (i, 0)),
            pl.BlockSpec((FLAT, C_IN), lambda i: (0, 0)),
            pl.BlockSpec((C_IN, FC_OUT), lambda i: (0, 0)),
            pl.BlockSpec((1, FC_OUT), lambda i: (0, 0)),
            pl.BlockSpec((FC_OUT, FC_OUT), lambda i: (0, 0)),
            pl.BlockSpec((1, FC_OUT), lambda i: (0, 0)),
            pl.BlockSpec((FC_OUT, M_REF), lambda i: (0, 0)),
            pl.BlockSpec((8, M_REF + 1), lambda i: (0, 0)),
            pl.BlockSpec((1, 2 * M_REF + 2), lambda i: (0, 0)),
        ],
        out_specs=[
            pl.BlockSpec((BN, M_REF), lambda i: (i, 0)),
            pl.BlockSpec((BN, M_REF + 1), lambda i: (i, 0)),
            pl.BlockSpec((2, BN, M_REF + 1), lambda i: (0, i, 0)),
            pl.BlockSpec((1, BN, 2 * M_REF + 2), lambda i: (0, i, 0)),
        ],
        out_shape=[
            jax.ShapeDtypeStruct((N_DET, M_REF), jnp.float32),
            jax.ShapeDtypeStruct((N_DET, M_REF + 1), jnp.float32),
            jax.ShapeDtypeStruct((2, N_DET, M_REF + 1), jnp.float32),
            jax.ShapeDtypeStruct((1, N_DET, 2 * M_REF + 2), jnp.float32),
        ],
        compiler_params=pltpu.CompilerParams(dimension_semantics=("arbitrary",)),
    )(x2, bboxes, p, W1, b1r, W2, b2r, rft, refg, rcil)

    return prod, ious2, dxy, dsp


# speedup vs baseline: 5.8812x; 5.8812x over previous
import jax
import jax.numpy as jnp
from jax.experimental import pallas as pl
from jax.experimental.pallas import tpu as pltpu

N_DET = 5000
M_REF = 1000
C_IN = 256
SPATIAL = 49
FC_OUT = 1024

BN = 200
BM = 200


def _pool_fc(xb, w1, b1, w2, b2):
    pooled = jnp.sum(xb, axis=0) / 49.0
    h = jnp.maximum(jnp.dot(pooled, w1, preferred_element_type=jnp.float32) + b1, 0.0)
    return jnp.dot(h, w2, preferred_element_type=jnp.float32) + b2


def _ref_kernel(rx_ref, w1_ref, b1_ref, w2_ref, b2_ref, rf_ref):
    rf_ref[...] = _pool_fc(rx_ref[...], w1_ref[...], b1_ref[...],
                           w2_ref[...], b2_ref[...])


def _main_kernel(x_ref, bb_ref, w1_ref, b1_ref, w2_ref, b2_ref,
                 rft_ref, refg_ref, rcil_ref,
                 prod_ref, ious_ref, dxy_ref, dsp_ref):
    xf = _pool_fc(x_ref[...], w1_ref[...], b1_ref[...], w2_ref[...], b2_ref[...])
    prod_ref[...] = jnp.dot(xf, rft_ref[...], preferred_element_type=jnp.float32)

    bb = bb_ref[...]
    x1 = bb[:, 0:1]
    y1 = bb[:, 1:2]
    x2 = bb[:, 2:3]
    y2 = bb[:, 3:4]
    cx = (x1 + x2) / 2.0
    cy = (y1 + y2) / 2.0
    rg = refg_ref[...]
    rx1 = rg[0:1, :]
    ry1 = rg[1:2, :]
    rx2 = rg[2:3, :]
    ry2 = rg[3:4, :]
    rcx = rg[4:5, :]
    rcy = rg[5:6, :]
    areab = rg[6:7, :]
    dx = rcx - cx
    dy = rcy - cy
    sx1 = x1 + dx
    sy1 = y1 + dy
    sx2 = x2 + dx
    sy2 = y2 + dy
    wx = jnp.maximum(jnp.minimum(sx2, rx2) - jnp.maximum(sx1, rx1), 0.0)
    wy = jnp.maximum(jnp.minimum(sy2, ry2) - jnp.maximum(sy1, ry1), 0.0)
    ov = wx * wy
    areaa = (sx2 - sx1) * (sy2 - sy1)
    union = areaa + areab - ov
    ious_ref[...] = ov / jnp.maximum(union, 1e-6)

    col = jax.lax.broadcasted_iota(jnp.int32, (BN, M_REF + 1), 1)
    keep = col >= 1
    dxy_ref[0] = jnp.where(keep, dx, 0.0)
    dxy_ref[1] = jnp.where(keep, dy, 0.0)

    col2 = jax.lax.broadcasted_iota(jnp.int32, (BN, 2 * M_REF + 2), 1)
    c_il = jnp.where((col2 & 1) == 0, cx, cy)
    dsp_ref[0] = jnp.where(col2 >= 2, rcil_ref[...] - c_il, 0.0)


def kernel(bboxes, ref_bboxes, x, ref_x, x_n, ref_x_n, W1, b1, W2, b2):
    del x_n, ref_x_n
    xt = jnp.transpose(x, (2, 3, 0, 1)).reshape(SPATIAL, N_DET, C_IN)
    rxt = jnp.transpose(ref_x, (2, 3, 0, 1)).reshape(SPATIAL, M_REF, C_IN)
    b1r = b1.reshape(1, FC_OUT)
    b2r = b2.reshape(1, FC_OUT)

    rf = pl.pallas_call(
        _ref_kernel,
        grid=(M_REF // BM,),
        in_specs=[
            pl.BlockSpec((SPATIAL, BM, C_IN), lambda i: (0, i, 0)),
            pl.BlockSpec((C_IN, FC_OUT), lambda i: (0, 0)),
            pl.BlockSpec((1, FC_OUT), lambda i: (0, 0)),
            pl.BlockSpec((FC_OUT, FC_OUT), lambda i: (0, 0)),
            pl.BlockSpec((1, FC_OUT), lambda i: (0, 0)),
        ],
        out_specs=pl.BlockSpec((BM, FC_OUT), lambda i: (i, 0)),
        out_shape=jax.ShapeDtypeStruct((M_REF, FC_OUT), jnp.float32),
        compiler_params=pltpu.CompilerParams(dimension_semantics=("arbitrary",)),
    )(rxt, W1, b1r, W2, b2r)
    rft = rf.T

    rcx = (ref_bboxes[:, 0] + ref_bboxes[:, 2]) / 2.0
    rcy = (ref_bboxes[:, 1] + ref_bboxes[:, 3]) / 2.0
    areab = (ref_bboxes[:, 2] - ref_bboxes[:, 0]) * (ref_bboxes[:, 3] - ref_bboxes[:, 1])
    refg = jnp.zeros((8, M_REF + 1), jnp.float32)
    refg = refg.at[0, 1:].set(ref_bboxes[:, 0])
    refg = refg.at[1, 1:].set(ref_bboxes[:, 1])
    refg = refg.at[2, 1:].set(ref_bboxes[:, 2])
    refg = refg.at[3, 1:].set(ref_bboxes[:, 3])
    refg = refg.at[4, 1:].set(rcx)
    refg = refg.at[5, 1:].set(rcy)
    refg = refg.at[6, 1:].set(areab)
    rcil = jnp.concatenate(
        [jnp.zeros((2,), jnp.float32), jnp.stack([rcx, rcy], axis=1).reshape(-1)]
    ).reshape(1, 2 * M_REF + 2)

    prod, ious2, dxy, dsp = pl.pallas_call(
        _main_kernel,
        grid=(N_DET // BN,),
        in_specs=[
            pl.BlockSpec((SPATIAL, BN, C_IN), lambda i: (0, i, 0)),
            pl.BlockSpec((BN, 4), lambda i: (i, 0)),
            pl.BlockSpec((C_IN, FC_OUT), lambda i: (0, 0)),
            pl.BlockSpec((1, FC_OUT), lambda i: (0, 0)),
            pl.BlockSpec((FC_OUT, FC_OUT), lambda i: (0, 0)),
            pl.BlockSpec((1, FC_OUT), lambda i: (0, 0)),
            pl.BlockSpec((FC_OUT, M_REF), lambda i: (0, 0)),
            pl.BlockSpec((8, M_REF + 1), lambda i: (0, 0)),
            pl.BlockSpec((1, 2 * M_REF + 2), lambda i: (0, 0)),
        ],
        out_specs=[
            pl.BlockSpec((BN, M_REF), lambda i: (i, 0)),
            pl.BlockSpec((BN, M_REF + 1), lambda i: (i, 0)),
            pl.BlockSpec((2, BN, M_REF + 1), lambda i: (0, i, 0)),
            pl.BlockSpec((1, BN, 2 * M_REF + 2), lambda i: (0, i, 0)),
        ],
        out_shape=[
            jax.ShapeDtypeStruct((N_DET, M_REF), jnp.float32),
            jax.ShapeDtypeStruct((N_DET, M_REF + 1), jnp.float32),
            jax.ShapeDtypeStruct((2, N_DET, M_REF + 1), jnp.float32),
            jax.ShapeDtypeStruct((1, N_DET, 2 * M_REF + 2), jnp.float32),
        ],
        compiler_params=pltpu.CompilerParams(dimension_semantics=("arbitrary",)),
    )(xt, bboxes, W1, b1r, W2, b2r, rft, refg, rcil)

    return prod, ious2, dxy, dsp


# speedup vs baseline: 6.2519x; 1.0630x over previous
import jax
import jax.numpy as jnp
from jax.experimental import pallas as pl
from jax.experimental.pallas import tpu as pltpu

N_DET = 5000
M_REF = 1000
C_IN = 256
SPATIAL = 49
FC_OUT = 1024

BN = 200
BM = 200


def _pool_fc(xb, w1, b1, w2, b2):
    pooled = jnp.sum(xb, axis=0) / 49.0
    h = jnp.maximum(jnp.dot(pooled, w1, preferred_element_type=jnp.float32) + b1, 0.0)
    return jnp.dot(h, w2, preferred_element_type=jnp.float32) + b2


def _ref_kernel(rx_ref, w1_ref, b1_ref, w2_ref, b2_ref, rf_ref):
    rf_ref[...] = _pool_fc(rx_ref[...], w1_ref[...], b1_ref[...],
                           w2_ref[...], b2_ref[...])


def _main_kernel(x_ref, bb_ref, w1_ref, b1_ref, w2_ref, b2_ref,
                 rft_ref, refg_ref, rcil_ref,
                 prod_ref, ious_ref, dxy_ref, dsp_ref):
    xf = _pool_fc(x_ref[...], w1_ref[...], b1_ref[...], w2_ref[...], b2_ref[...])
    prod_ref[...] = jax.lax.dot_general(
        xf, rft_ref[...], (((1,), (1,)), ((), ())),
        preferred_element_type=jnp.float32)

    bb = bb_ref[...]
    x1 = bb[:, 0:1]
    y1 = bb[:, 1:2]
    x2 = bb[:, 2:3]
    y2 = bb[:, 3:4]
    cx = (x1 + x2) / 2.0
    cy = (y1 + y2) / 2.0
    rg = refg_ref[...]
    rx1 = rg[0:1, :]
    ry1 = rg[1:2, :]
    rx2 = rg[2:3, :]
    ry2 = rg[3:4, :]
    rcx = rg[4:5, :]
    rcy = rg[5:6, :]
    areab = rg[6:7, :]
    dx = rcx - cx
    dy = rcy - cy
    sx1 = x1 + dx
    sy1 = y1 + dy
    sx2 = x2 + dx
    sy2 = y2 + dy
    wx = jnp.maximum(jnp.minimum(sx2, rx2) - jnp.maximum(sx1, rx1), 0.0)
    wy = jnp.maximum(jnp.minimum(sy2, ry2) - jnp.maximum(sy1, ry1), 0.0)
    ov = wx * wy
    areaa = (sx2 - sx1) * (sy2 - sy1)
    union = areaa + areab - ov
    ious_ref[...] = ov / jnp.maximum(union, 1e-6)

    col = jax.lax.broadcasted_iota(jnp.int32, (BN, M_REF + 1), 1)
    keep = col >= 1
    dxy_ref[0] = jnp.where(keep, dx, 0.0)
    dxy_ref[1] = jnp.where(keep, dy, 0.0)

    col2 = jax.lax.broadcasted_iota(jnp.int32, (BN, 2 * M_REF + 2), 1)
    c_il = jnp.where((col2 & 1) == 0, cx, cy)
    dsp_ref[0] = jnp.where(col2 >= 2, rcil_ref[...] - c_il, 0.0)


def kernel(bboxes, ref_bboxes, x, ref_x, x_n, ref_x_n, W1, b1, W2, b2):
    del x_n, ref_x_n
    xt = jnp.transpose(x, (2, 3, 0, 1)).reshape(SPATIAL, N_DET, C_IN)
    rxt = jnp.transpose(ref_x, (2, 3, 0, 1)).reshape(SPATIAL, M_REF, C_IN)
    b1r = b1.reshape(1, FC_OUT)
    b2r = b2.reshape(1, FC_OUT)

    rf = pl.pallas_call(
        _ref_kernel,
        grid=(M_REF // BM,),
        in_specs=[
            pl.BlockSpec((SPATIAL, BM, C_IN), lambda i: (0, i, 0)),
            pl.BlockSpec((C_IN, FC_OUT), lambda i: (0, 0)),
            pl.BlockSpec((1, FC_OUT), lambda i: (0, 0)),
            pl.BlockSpec((FC_OUT, FC_OUT), lambda i: (0, 0)),
            pl.BlockSpec((1, FC_OUT), lambda i: (0, 0)),
        ],
        out_specs=pl.BlockSpec((BM, FC_OUT), lambda i: (i, 0)),
        out_shape=jax.ShapeDtypeStruct((M_REF, FC_OUT), jnp.float32),
        compiler_params=pltpu.CompilerParams(dimension_semantics=("arbitrary",)),
    )(rxt, W1, b1r, W2, b2r)

    rcx = (ref_bboxes[:, 0] + ref_bboxes[:, 2]) / 2.0
    rcy = (ref_bboxes[:, 1] + ref_bboxes[:, 3]) / 2.0
    areab = (ref_bboxes[:, 2] - ref_bboxes[:, 0]) * (ref_bboxes[:, 3] - ref_bboxes[:, 1])
    refg = jnp.pad(
        jnp.stack([ref_bboxes[:, 0], ref_bboxes[:, 1], ref_bboxes[:, 2],
                   ref_bboxes[:, 3], rcx, rcy, areab,
                   jnp.zeros((M_REF,), jnp.float32)], axis=0),
        ((0, 0), (1, 0)))
    rcil = jnp.concatenate(
        [jnp.zeros((2,), jnp.float32), jnp.stack([rcx, rcy], axis=1).reshape(-1)]
    ).reshape(1, 2 * M_REF + 2)

    prod, ious2, dxy, dsp = pl.pallas_call(
        _main_kernel,
        grid=(N_DET // BN,),
        in_specs=[
            pl.BlockSpec((SPATIAL, BN, C_IN), lambda i: (0, i, 0)),
            pl.BlockSpec((BN, 4), lambda i: (i, 0)),
            pl.BlockSpec((C_IN, FC_OUT), lambda i: (0, 0)),
            pl.BlockSpec((1, FC_OUT), lambda i: (0, 0)),
            pl.BlockSpec((FC_OUT, FC_OUT), lambda i: (0, 0)),
            pl.BlockSpec((1, FC_OUT), lambda i: (0, 0)),
            pl.BlockSpec((M_REF, FC_OUT), lambda i: (0, 0)),
            pl.BlockSpec((8, M_REF + 1), lambda i: (0, 0)),
            pl.BlockSpec((1, 2 * M_REF + 2), lambda i: (0, 0)),
        ],
        out_specs=[
            pl.BlockSpec((BN, M_REF), lambda i: (i, 0)),
            pl.BlockSpec((BN, M_REF + 1), lambda i: (i, 0)),
            pl.BlockSpec((2, BN, M_REF + 1), lambda i: (0, i, 0)),
            pl.BlockSpec((1, BN, 2 * M_REF + 2), lambda i: (0, i, 0)),
        ],
        out_shape=[
            jax.ShapeDtypeStruct((N_DET, M_REF), jnp.float32),
            jax.ShapeDtypeStruct((N_DET, M_REF + 1), jnp.float32),
            jax.ShapeDtypeStruct((2, N_DET, M_REF + 1), jnp.float32),
            jax.ShapeDtypeStruct((1, N_DET, 2 * M_REF + 2), jnp.float32),
        ],
        compiler_params=pltpu.CompilerParams(dimension_semantics=("arbitrary",)),
    )(xt, bboxes, W1, b1r, W2, b2r, rf, refg, rcil)

    return prod, ious2, dxy, dsp
